# baseline (speedup 1.0000x reference)
import functools

import jax
import jax.numpy as jnp
from jax import lax
from jax.experimental import pallas as pl
from jax.experimental.pallas import tpu as pltpu
from jax.experimental.pallas import tpu_sc as plsc

NC, NS, L = 2, 16, 16
NW = NC * NS

_LN2 = 0.6931471805599453
_SQRT2 = 1.4142135623730951


def _ln(x):
    bits = plsc.bitcast(x, jnp.int32)
    e = ((bits >> 23) & 0xFF) - 127
    m = plsc.bitcast((bits & 0x7FFFFF) | (127 << 23), jnp.float32)
    ef = e.astype(jnp.float32)
    big = m > _SQRT2
    m = jnp.where(big, m * 0.5, m)
    ef = jnp.where(big, ef + 1.0, ef)
    z = (m - 1.0) / (m + 1.0)
    z2 = z * z
    p = 1.0 / 11.0
    p = p * z2 + 1.0 / 9.0
    p = p * z2 + 1.0 / 7.0
    p = p * z2 + 1.0 / 5.0
    p = p * z2 + 1.0 / 3.0
    p = p * z2 + 1.0
    return ef * _LN2 + 2.0 * z * p


def _make_sc_partials(B, D, interpret=False):
    bpw = B // NW
    nchunks = bpw // L
    mesh = plsc.VectorSubcoreMesh(
        core_axis_name="c", subcore_axis_name="s", num_cores=NC, num_subcores=NS
    )

    @functools.partial(
        pl.kernel,
        out_type=jax.ShapeDtypeStruct((NW, L), jnp.float32),
        mesh=mesh,
        scratch_types=[
            pltpu.VMEM((bpw,), jnp.int32),
            pltpu.VMEM((bpw,), jnp.int32),
            pltpu.VMEM((bpw, D), jnp.float32),
            pltpu.VMEM((bpw, D), jnp.float32),
            pltpu.VMEM((bpw,), jnp.float32),
            pltpu.VMEM((bpw,), jnp.float32),
            pltpu.VMEM((bpw,), jnp.float32),
            pltpu.VMEM((bpw,), jnp.float32),
            pltpu.VMEM((1, L), jnp.float32),
            pltpu.SemaphoreType.DMA,
        ],
        compiler_params=pltpu.CompilerParams(
            needs_layout_passes=False, use_tc_tiling_on_sc=False
        ),
        interpret=interpret,
    )
    def sc_partials(i_hbm, j_hbm, co_hbm, wt_hbm, v_hbm, w_hbm, bv_hbm, bw_hbm,
                    out_hbm, idx_i, idx_j, rows_v, rows_w, bi, bj, co_v, wt_v,
                    part, sem):
        cid = lax.axis_index("c")
        sid = lax.axis_index("s")
        wid = sid * NC + cid
        base = wid * bpw

        pltpu.sync_copy(i_hbm.at[pl.ds(base, bpw)], idx_i)
        pltpu.sync_copy(j_hbm.at[pl.ds(base, bpw)], idx_j)
        pltpu.sync_copy(co_hbm.at[pl.ds(base, bpw)], co_v)
        pltpu.sync_copy(wt_hbm.at[pl.ds(base, bpw)], wt_v)

        cp1 = pltpu.async_copy(v_hbm.at[idx_i], rows_v, sem)
        cp2 = pltpu.async_copy(w_hbm.at[idx_j], rows_w, sem)
        cp3 = pltpu.async_copy(bv_hbm.at[idx_i], bi, sem)
        cp4 = pltpu.async_copy(bw_hbm.at[idx_j], bj, sem)
        cp1.wait()
        cp2.wait()
        cp3.wait()
        cp4.wait()

        iota = lax.broadcasted_iota(jnp.int32, (L,), 0)
        zerosf = jnp.zeros((L,), jnp.float32)
        acc_s1 = zerosf
        acc_s2 = zerosf
        acc_s3 = zerosf
        acc_t1 = zerosf
        acc_t2 = zerosf
        for chunk in range(nchunks):
            ridx = iota + chunk * L

            def dot_body(d, acc):
                col = jnp.full((L,), d, jnp.int32)
                va = plsc.load_gather(rows_v, [ridx, col])
                vb = plsc.load_gather(rows_w, [ridx, col])
                return acc + va * vb

            sim = lax.fori_loop(0, D, dot_body, zerosf)
            co_c = co_v[pl.ds(chunk * L, L)]
            wt_c = wt_v[pl.ds(chunk * L, L)]
            bi_c = bi[pl.ds(chunk * L, L)]
            bj_c = bj[pl.ds(chunk * L, L)]
            a = sim - _ln(co_c)
            wa = wt_c * a
            acc_s1 = acc_s1 + wa * a
            acc_s2 = acc_s2 + wa
            acc_s3 = acc_s3 + wt_c
            t = bi_c + bj_c
            acc_t1 = acc_t1 + t
            acc_t2 = acc_t2 + t * t

        s1 = jnp.sum(acc_s1)
        s2 = jnp.sum(acc_s2)
        s3 = jnp.sum(acc_s3)
        t1 = jnp.sum(acc_t1)
        t2 = jnp.sum(acc_t2)
        outv = jnp.where(iota == 0, s1, 0.0)
        outv = outv + jnp.where(iota == 1, s2, 0.0)
        outv = outv + jnp.where(iota == 2, s3, 0.0)
        outv = outv + jnp.where(iota == 3, t1, 0.0)
        outv = outv + jnp.where(iota == 4, t2, 0.0)
        part[0, :] = outv
        pltpu.sync_copy(part, out_hbm.at[pl.ds(wid, 1)])

    return sc_partials


def kernel(i, j, co_occur, weight, V, W, BV, BW):
    B = i.shape[0]
    D = V.shape[1]
    p = _make_sc_partials(B, D)(
        i, j, co_occur, weight, V, W,
        jnp.reshape(BV, (-1,)), jnp.reshape(BW, (-1,))
    )
    s1 = jnp.sum(p[:, 0])
    s2 = jnp.sum(p[:, 1])
    s3 = jnp.sum(p[:, 2])
    t1 = jnp.sum(p[:, 3])
    t2 = jnp.sum(p[:, 4])
    return 0.5 * (B * s1 + 2.0 * t1 * s2 + t2 * s3)
